# baseline (speedup 1.0000x reference)
import functools

import jax
import jax.numpy as jnp
from jax import lax
from jax.experimental import pallas as pl
from jax.experimental.pallas import tpu as pltpu
from jax.experimental.pallas import tpu_sc as plsc

N_PMTS = 12
N_TICKS = 550
TP = 560
T = 8192
B = 8
NC = 2
NS = 16
NW = NC * NS
CHUNK = T // NW
W = 6
ACC = B * 16 * TP


def _mlp_body(e_ref, u_ref, w1_ref, b1_ref, w2_ref, b2_ref, sc_ref, nl_ref,
              out_ref):
    x = e_ref[:, 0:1]
    y = e_ref[:, 1:2]
    z = e_ref[:, 2:3]
    h = jax.nn.sigmoid(x * w1_ref[0:1, :] + y * w1_ref[1:2, :] + b1_ref[...])
    pmt = jax.nn.sigmoid(
        jnp.dot(h, w2_ref[...], preferred_element_type=jnp.float32)
        + b2_ref[...])
    prob = 1.0 - jnp.exp(z * nl_ref[...])
    mask = (prob > u_ref[...]).astype(jnp.float32)
    out_ref[...] = pmt * sc_ref[...] * mask


def _mlp_pmt(electrons, u2, w1, b1, w2p, b2p, scale16, neg_inv_lt):
    return pl.pallas_call(
        _mlp_body,
        out_shape=jax.ShapeDtypeStruct((T, 16), jnp.float32),
    )(electrons, u2, w1, b1, w2p, b2p, scale16, neg_inv_lt)


def _sc_scatter_body(z_hbm, pmt_hbm, cu_hbm, zero_hbm, out_hbm,
                     z_v, pmt_v, cu_v, ib_v, t0_v, acc_v, sem):
    wid = lax.axis_index("s") * NC + lax.axis_index("c")
    base = wid * CHUNK
    pltpu.sync_copy(z_hbm.at[pl.ds(base, CHUNK)], z_v.at[pl.ds(0, CHUNK)])
    pltpu.sync_copy(pmt_hbm.at[pl.ds(base, CHUNK)], pmt_v)
    pltpu.sync_copy(cu_hbm, cu_v)
    pltpu.sync_copy(zero_hbm, acc_v)

    iota_i = lax.iota(jnp.int32, 16)
    iota_f = iota_i.astype(jnp.float32)
    iota560 = iota_i * TP

    cuts = [plsc.load_gather(cu_v, [jnp.full((16,), s, dtype=jnp.int32)])
            for s in range(1, B)]

    for g in range(CHUNK // 16):
        sl = pl.ds(g * 16, 16)
        zg = z_v[sl]
        t0i = jnp.clip(zg.astype(jnp.int32) - 2, 0, TP - W)
        t0_v[sl] = t0i.astype(jnp.float32)
        tvec = jnp.full((16,), base + g * 16, dtype=jnp.int32) + iota_i
        seg = jnp.zeros((16,), dtype=jnp.int32)
        for c in cuts:
            seg = seg + (tvec >= c).astype(jnp.int32)
        ib_v[sl] = seg * (16 * TP) + t0i

    def body(e, _):
        efull = jnp.full((16,), e, dtype=jnp.int32)
        zb = plsc.load_gather(z_v, [efull])
        t0 = plsc.load_gather(t0_v, [efull])
        ib = plsc.load_gather(ib_v, [efull])
        pm = plsc.load_gather(pmt_v, [efull, iota_i])
        c0 = t0 + (0.5 - zb)
        idxv = ib + iota560
        for w in range(W):
            dw = c0 + jnp.float32(w)
            gv = jnp.exp(dw * dw * -10.0)
            plsc.addupdate_scatter(acc_v, [idxv + w], pm * gv)
        return ()

    lax.fori_loop(0, CHUNK, body, (), unroll=False)
    pltpu.sync_copy(acc_v, out_hbm.at[wid])


@functools.partial(jax.jit, static_argnames=())
def _sc_scatter(zs, pmt16, cu16, zero_acc):
    mesh = plsc.VectorSubcoreMesh(core_axis_name="c", subcore_axis_name="s")
    kfn = pl.kernel(
        _sc_scatter_body,
        out_type=jax.ShapeDtypeStruct((NW, ACC), jnp.float32),
        mesh=mesh,
        scratch_types=[
            pltpu.VMEM((CHUNK + 16,), jnp.float32),
            pltpu.VMEM((CHUNK, 16), jnp.float32),
            pltpu.VMEM((16,), jnp.int32),
            pltpu.VMEM((CHUNK + 16,), jnp.int32),
            pltpu.VMEM((CHUNK + 16,), jnp.float32),
            pltpu.VMEM((ACC,), jnp.float32),
            pltpu.SemaphoreType.DMA,
        ],
        compiler_params=pltpu.CompilerParams(needs_layout_passes=False),
    )
    return kfn(zs, pmt16, cu16, zero_acc)


def kernel(electrons, u, W1, b1, W2, b2, pmt_response_scale, lifetime,
           cu_seqlens):
    gauss_norm = 1.0 / (0.1 * 2.5066282746)
    w2p = jnp.concatenate([W2, jnp.zeros((28, 4), jnp.float32)], axis=1)
    b2p = jnp.concatenate([b2, jnp.full((4,), -40.0, jnp.float32)])
    sc16 = jnp.concatenate(
        [pmt_response_scale**2 * gauss_norm, jnp.zeros((4,), jnp.float32)])
    scale16 = sc16.reshape(1, 16)
    neg_inv_lt = (-1.0 / lifetime).reshape(1, 1).astype(jnp.float32)
    u2 = u.reshape(T, 1)

    pmt16 = _mlp_pmt(electrons, u2, W1, b1, w2p, b2p, scale16, neg_inv_lt)

    zs = electrons[:, 2]
    cu16 = jnp.concatenate(
        [cu_seqlens, jnp.full((16 - B - 1,), T, jnp.int32)]).astype(jnp.int32)
    zero_acc = jnp.zeros((ACC,), jnp.float32)

    partials = _sc_scatter(zs, pmt16, cu16, zero_acc)
    out = partials.sum(axis=0).reshape(B, 16, TP)
    return out[:, :N_PMTS, :N_TICKS]
